# baseline (speedup 1.0000x reference)
import functools
import math

import jax
import jax.numpy as jnp
from jax import lax
from jax.experimental import pallas as pl
from jax.experimental.pallas import tpu as pltpu
from jax.experimental.pallas import tpu_sc as plsc

_NUM_CORES = 2
_NUM_SUBCORES = 16
_NW = _NUM_CORES * _NUM_SUBCORES
_LANES = 16
_CHUNK = 128


def _make_lookup(vocab, d_model, n_idx, scale):
    per_w = n_idx // _NW
    n_chunk = per_w // _CHUNK
    mesh = plsc.VectorSubcoreMesh(core_axis_name="c", subcore_axis_name="s")

    @functools.partial(
        pl.kernel,
        mesh=mesh,
        out_type=jax.ShapeDtypeStruct((n_idx, d_model), jnp.float32),
        scratch_types=[
            pltpu.VMEM((n_chunk, _CHUNK), jnp.int32),
            pltpu.VMEM((_CHUNK, d_model), jnp.float32),
            pltpu.SemaphoreType.DMA,
        ],
        compiler_params=pltpu.CompilerParams(use_tc_tiling_on_sc=False),
    )
    def lookup(table_hbm, idx_hbm, out_hbm, idx_v, rows_v, sem):
        wid = lax.axis_index("s") * _NUM_CORES + lax.axis_index("c")
        pltpu.sync_copy(idx_hbm.at[wid], idx_v)

        def chunk_body(j, carry):
            pltpu.async_copy(table_hbm.at[idx_v.at[j]], rows_v, sem).wait()

            def row_body(i, c):
                for t in range(d_model // _LANES):
                    sl = pl.ds(t * _LANES, _LANES)
                    rows_v[i, sl] = rows_v[i, sl] * scale
                return c

            lax.fori_loop(0, _CHUNK, row_body, 0)

            row_base = (wid * n_chunk + j) * _CHUNK
            pltpu.sync_copy(rows_v, out_hbm.at[pl.ds(row_base, _CHUNK)])
            return carry

        lax.fori_loop(0, n_chunk, chunk_body, 0)

    return lookup


def kernel(X, table):
    b, s = X.shape
    vocab, d_model = table.shape
    n_idx = b * s
    scale = math.sqrt(d_model)
    idx = X.reshape(_NW, (n_idx // _NW) // _CHUNK, _CHUNK).astype(jnp.int32)
    out = _make_lookup(vocab, d_model, n_idx, scale)(table, idx)
    return out.reshape(b, s, d_model)


# speedup vs baseline: 1.2069x; 1.2069x over previous
import functools
import math

import jax
import jax.numpy as jnp
from jax import lax
from jax.experimental import pallas as pl
from jax.experimental.pallas import tpu as pltpu
from jax.experimental.pallas import tpu_sc as plsc

_NUM_CORES = 2
_NUM_SUBCORES = 16
_NW = _NUM_CORES * _NUM_SUBCORES
_LANES = 16
_CHUNK = 128
_NG = 4
_NO = 2


def _make_lookup(d_model, n_idx, scale):
    per_w = n_idx // _NW
    n_chunk = per_w // _CHUNK
    n_outer = n_chunk // _NG
    n_vec = d_model // _LANES
    mesh = plsc.VectorSubcoreMesh(core_axis_name="c", subcore_axis_name="s")

    @functools.partial(
        pl.kernel,
        mesh=mesh,
        out_type=jax.ShapeDtypeStruct((n_idx, d_model), jnp.float32),
        scratch_types=[
            pltpu.VMEM((n_chunk, _CHUNK), jnp.int32),
            pltpu.VMEM((_NG, _CHUNK, d_model), jnp.float32),
            pltpu.VMEM((_NO, _CHUNK, d_model), jnp.float32),
        ]
        + [pltpu.SemaphoreType.DMA] * (_NG + _NO),
        compiler_params=pltpu.CompilerParams(use_tc_tiling_on_sc=False),
    )
    def lookup(table_hbm, idx_hbm, out_hbm, idx_v, gbuf, obuf, *sems):
        gsems = sems[:_NG]
        ssems = sems[_NG:]
        wid = lax.axis_index("s") * _NUM_CORES + lax.axis_index("c")
        pltpu.sync_copy(idx_hbm.at[wid], idx_v)

        def gather_desc(j, b):
            return pltpu.make_async_copy(
                table_hbm.at[idx_v.at[j]], gbuf.at[b], gsems[b]
            )

        def scatter_desc(j, bs):
            base = (wid * n_chunk + j) * _CHUNK
            return pltpu.make_async_copy(
                obuf.at[bs], out_hbm.at[pl.ds(base, _CHUNK)], ssems[bs]
            )

        for b in range(_NG):
            gather_desc(jnp.int32(b), b).start()

        def outer(g, carry):
            for b in range(_NG):
                j = g * _NG + b
                gather_desc(j, b).wait()
                bs = b % _NO
                if b < _NO:

                    @pl.when(g > 0)
                    def _():
                        scatter_desc(j, bs).wait()

                else:
                    scatter_desc(j, bs).wait()

                def row_body(i, c, b=b, bs=bs):
                    for k in range(4):
                        r = i * 4 + k
                        for t in range(n_vec):
                            sl = pl.ds(t * _LANES, _LANES)
                            obuf[bs, r, sl] = gbuf[b, r, sl] * scale
                    return c

                lax.fori_loop(0, _CHUNK // 4, row_body, 0)
                scatter_desc(j, bs).start()

                @pl.when(g < n_outer - 1)
                def _(b=b, j=j):
                    gather_desc(j + _NG, b).start()

            return carry

        lax.fori_loop(0, n_outer, outer, 0)
        for bs in range(_NO):
            scatter_desc(jnp.int32(n_chunk - _NO + bs), bs).wait()

    return lookup


def kernel(X, table):
    b, s = X.shape
    _, d_model = table.shape
    n_idx = b * s
    scale = math.sqrt(d_model)
    idx = X.reshape(_NW, (n_idx // _NW) // _CHUNK, _CHUNK).astype(jnp.int32)
    out = _make_lookup(d_model, n_idx, scale)(table, idx)
    return out.reshape(b, s, d_model)
